# baseline (speedup 1.0000x reference)
import functools

import jax
import jax.numpy as jnp
from jax import lax
from jax.experimental import pallas as pl
from jax.experimental.pallas import tpu as pltpu

_EPS = 1e-5


def _bn_scale_shift(g, b, m, v):
    s = g * lax.rsqrt(v + _EPS)
    return s.astype(jnp.float32), (b - m * s).astype(jnp.float32)


def _dot_chunked(a, w, tk=512):
    K = a.shape[1]
    if K <= tk:
        return jnp.dot(a, w, preferred_element_type=jnp.float32)
    acc = jnp.dot(a[:, :tk], w[:tk], preferred_element_type=jnp.float32)
    for k0 in range(tk, K, tk):
        acc = acc + jnp.dot(a[:, k0:k0 + tk], w[k0:k0 + tk],
                            preferred_element_type=jnp.float32)
    return acc


def _dot_fori(lhs_slice, w_ref, M, N, K, tk):
    def body(k, acc):
        k0 = k * tk
        return acc + jnp.dot(lhs_slice(k0), w_ref[pl.ds(k0, tk)],
                             preferred_element_type=jnp.float32)
    return lax.fori_loop(0, K // tk, body, jnp.zeros((M, N), jnp.float32))


_CONV2_TK = {64: 128, 128: 384, 256: 384, 512: 512}


def _tap_s2(arr, i, j, Ho, Wo):
    nb, Hp, Wp, C = arr.shape
    r = arr.reshape(nb, Hp // 2, 2, Wp // 2, 2, C)
    return r[:, i // 2:i // 2 + Ho, i % 2, j // 2:j // 2 + Wo, j % 2, :]


def _stem_kernel(x_ref, w_ref, s_ref, t_ref, o_ref, scr_ref):
    v = x_ref[0]
    for i in range(7):
        hs = v[i // 2:i // 2 + 112]
        p6 = 6 * (i % 2)
        for k in range(4):
            w6 = min(6, 21 - 6 * k)
            piece = hs[:, k:k + 112, p6:p6 + w6].reshape(12544, w6)
            scr_ref[:, 21 * i + 6 * k:21 * i + 6 * k + w6] = piece
    scr_ref[:, 147:] = jnp.zeros((12544, 256 - 147), jnp.bfloat16)
    y = jnp.dot(scr_ref[...], w_ref[...], preferred_element_type=jnp.float32)
    y = jnp.maximum(y * s_ref[...] + t_ref[...], 0.0).astype(jnp.bfloat16)
    y = y.reshape(112, 112, 64)
    yp = jnp.pad(y, ((1, 1), (1, 1), (0, 0)),
                 constant_values=jnp.bfloat16(-jnp.inf))
    m = None
    for i in range(3):
        for j in range(3):
            tap = _tap_s2(yp[None], i, j, 56, 56)
            m = tap if m is None else jnp.maximum(m, tap)
    o_ref[...] = m


def _stem(x, w, g, b, mn, vr):
    N = x.shape[0]
    xf = x.reshape(N, 224, 672)
    xf = jnp.pad(xf, ((0, 0), (3, 3), (9, 9)))
    xf = (xf.reshape(N, 115, 2, 115, 6)
            .transpose(0, 1, 3, 2, 4)
            .reshape(N, 115, 115, 12))
    s, t = _bn_scale_shift(g, b, mn, vr)
    return pl.pallas_call(
        _stem_kernel,
        out_shape=jax.ShapeDtypeStruct((N, 56, 56, 64), jnp.bfloat16),
        grid=(N,),
        in_specs=[pl.BlockSpec((1, 115, 115, 12), lambda n: (n, 0, 0, 0)),
                  pl.BlockSpec((256, 64), lambda n: (0, 0)),
                  pl.BlockSpec((1, 64), lambda n: (0, 0)),
                  pl.BlockSpec((1, 64), lambda n: (0, 0))],
        out_specs=pl.BlockSpec((1, 56, 56, 64), lambda n: (n, 0, 0, 0)),
        scratch_shapes=[pltpu.VMEM((12544, 256), jnp.bfloat16)],
        compiler_params=pltpu.CompilerParams(
            dimension_semantics=("parallel",),
            vmem_limit_bytes=100 * 1024 * 1024),
    )(xf, w, s, t)


def _make_block_kernel(nb, H, W, cin, planes, stride, down):
    Ho, Wo = H // stride, W // stride
    M1 = nb * H * W
    Mo = nb * Ho * Wo
    cout = 4 * planes

    tk2 = _CONV2_TK.get(planes, 9 * planes)
    Kp2 = -(-9 * planes // tk2) * tk2

    def kern(x_ref, w1_ref, s1_ref, t1_ref, w2_ref, s2_ref, t2_ref,
             w3_ref, s3_ref, t3_ref, *rest):
        if down:
            wd_ref, sd_ref, td_ref, o_ref, scr_ref = rest
        else:
            o_ref, scr_ref = rest
        x4 = x_ref[...]
        x = x4.reshape(M1, cin)

        if cin > 512:
            u = _dot_fori(
                lambda k0: x_ref[:, :, :, pl.ds(k0, 512)].reshape(M1, 512),
                w1_ref, M1, planes, cin, 512)
        else:
            u = jnp.dot(x, w1_ref[...], preferred_element_type=jnp.float32)
        u = jnp.maximum(u * s1_ref[...] + t1_ref[...], 0.0).astype(jnp.bfloat16)

        up = jnp.pad(u.reshape(nb, H, W, planes),
                     ((0, 0), (1, 1), (1, 1), (0, 0)))
        for i in range(3):
            for j in range(3):
                if stride == 1:
                    a = up[:, i:i + Ho, j:j + Wo, :]
                else:
                    a = _tap_s2(up, i, j, Ho, Wo)
                t = i * 3 + j
                scr_ref[:, t * planes:(t + 1) * planes] = a.reshape(Mo, planes)
        if Kp2 > 9 * planes:
            scr_ref[:, 9 * planes:] = jnp.zeros((Mo, Kp2 - 9 * planes),
                                                jnp.bfloat16)
        acc = _dot_fori(lambda k0: scr_ref[:, pl.ds(k0, tk2)],
                        w2_ref, Mo, planes, Kp2, tk2)
        v = jnp.maximum(acc * s2_ref[...] + t2_ref[...], 0.0).astype(jnp.bfloat16)

        y = jnp.dot(v, w3_ref[...], preferred_element_type=jnp.float32)
        y = (y * s3_ref[...] + t3_ref[...]).astype(jnp.bfloat16)

        if down:
            if cin > 512:
                def lhsd(k0):
                    v = x_ref[:, :, :, pl.ds(k0, 512)]
                    if stride != 1:
                        v = _tap_s2(v, 0, 0, Ho, Wo)
                    return v.reshape(Mo, 512)
                idn = _dot_fori(lhsd, wd_ref, Mo, cout, cin, 512)
            else:
                if stride == 1:
                    xs = x
                else:
                    xs = _tap_s2(x4, 0, 0, Ho, Wo).reshape(Mo, cin)
                idn = jnp.dot(xs, wd_ref[...],
                              preferred_element_type=jnp.float32)
            idn = (idn * sd_ref[...] + td_ref[...]).astype(jnp.bfloat16)
        else:
            idn = x.reshape(Mo, cout)

        out = jnp.maximum(y.astype(jnp.float32) + idn.astype(jnp.float32), 0.0)
        o_ref[...] = out.astype(jnp.bfloat16).reshape(nb, Ho, Wo, cout)

    return kern


def _bottleneck(x, p, stride, nb):
    N, H, W, cin = x.shape
    w1 = p["conv1"]
    planes = w1.shape[1]
    cout = 4 * planes
    Ho, Wo = H // stride, W // stride
    down = "down_conv" in p

    s1, t1 = _bn_scale_shift(*p["bn1"])
    s2, t2 = _bn_scale_shift(*p["bn2"])
    s3, t3 = _bn_scale_shift(*p["bn3"])
    w2 = p["conv2"]
    Kp2 = w2.shape[0]
    w3 = p["conv3"]

    c = pl.BlockSpec
    in_specs = [
        c((nb, H, W, cin), lambda n: (n, 0, 0, 0)),
        c((cin, planes), lambda n: (0, 0)),
        c((1, planes), lambda n: (0, 0)),
        c((1, planes), lambda n: (0, 0)),
        c((Kp2, planes), lambda n: (0, 0)),
        c((1, planes), lambda n: (0, 0)),
        c((1, planes), lambda n: (0, 0)),
        c((planes, cout), lambda n: (0, 0)),
        c((1, cout), lambda n: (0, 0)),
        c((1, cout), lambda n: (0, 0)),
    ]
    args = [x, w1, s1, t1, w2, s2, t2, w3, s3, t3]
    if down:
        sd, td = _bn_scale_shift(*p["down_bn"])
        in_specs += [c((cin, cout), lambda n: (0, 0)),
                     c((1, cout), lambda n: (0, 0)),
                     c((1, cout), lambda n: (0, 0))]
        args += [p["down_conv"], sd, td]

    kern = _make_block_kernel(nb, H, W, cin, planes, stride, down)
    return pl.pallas_call(
        kern,
        out_shape=jax.ShapeDtypeStruct((N, Ho, Wo, cout), jnp.bfloat16),
        grid=(N // nb,),
        in_specs=in_specs,
        out_specs=c((nb, Ho, Wo, cout), lambda n: (n, 0, 0, 0)),
        scratch_shapes=[pltpu.VMEM((nb * Ho * Wo, Kp2), jnp.bfloat16)],
        compiler_params=pltpu.CompilerParams(
            dimension_semantics=("parallel",),
            vmem_limit_bytes=100 * 1024 * 1024),
    )(*args)


def _head_kernel(f_ref, g_ref, b_ref, w_ref, cls_ref, feat_ref, scr_ref):
    f = f_ref[...].astype(jnp.float32)
    gfeat = jnp.mean(f, axis=1)
    mean = jnp.mean(gfeat, axis=0, keepdims=True)
    var = jnp.mean(jnp.square(gfeat - mean), axis=0, keepdims=True)
    bnf = (gfeat - mean) * lax.rsqrt(var + _EPS) * g_ref[...] + b_ref[...]
    scr_ref[...] = bnf.astype(jnp.bfloat16)
    N, C = scr_ref.shape
    cls_ref[...] = _dot_fori(lambda k0: scr_ref[:, pl.ds(k0, 512)],
                             w_ref, N, w_ref.shape[1], C, 512)
    feat_ref[...] = gfeat


def _head(feat, g, b, w):
    N = feat.shape[0]
    C = feat.shape[3]
    nc = w.shape[1]
    f3 = feat.reshape(N, 49, C)
    return pl.pallas_call(
        _head_kernel,
        out_shape=(jax.ShapeDtypeStruct((N, nc), jnp.float32),
                   jax.ShapeDtypeStruct((N, C), jnp.float32)),
        grid=(1,),
        in_specs=[pl.BlockSpec((N, 49, C), lambda i: (0, 0, 0)),
                  pl.BlockSpec((1, C), lambda i: (0, 0)),
                  pl.BlockSpec((1, C), lambda i: (0, 0)),
                  pl.BlockSpec((C, nc), lambda i: (0, 0))],
        out_specs=[pl.BlockSpec((N, nc), lambda i: (0, 0)),
                   pl.BlockSpec((N, C), lambda i: (0, 0))],
        scratch_shapes=[pltpu.VMEM((N, C), jnp.bfloat16)],
        compiler_params=pltpu.CompilerParams(
            vmem_limit_bytes=100 * 1024 * 1024),
    )(f3, g, b, w)


_STAGES = [(64, 3, 1), (128, 4, 2), (256, 6, 2), (512, 3, 2)]
_NB = {(0, True): 2, (0, False): 2,
       (1, True): 2, (1, False): 4,
       (2, True): 4, (2, False): 8,
       (3, True): 8, (3, False): 8}


@jax.jit
def _forward(P):
    x = jnp.transpose(P["x"], (0, 2, 3, 1)).astype(jnp.bfloat16)
    x = _stem(x, P["bb_conv1"], P["bb_bn1_g"], P["bb_bn1_b"],
              P["bb_bn1_m"], P["bb_bn1_v"])
    for S, (planes, nblocks, stride) in enumerate(_STAGES):
        for B in range(nblocks):
            s = stride if B == 0 else 1
            pfx = "s%db%d_" % (S, B)
            blk = {
                "conv1": P[pfx + "conv1"],
                "bn1": (P[pfx + "bn1_g"], P[pfx + "bn1_b"],
                        P[pfx + "bn1_m"], P[pfx + "bn1_v"]),
                "conv2": P[pfx + "conv2"],
                "bn2": (P[pfx + "bn2_g"], P[pfx + "bn2_b"],
                        P[pfx + "bn2_m"], P[pfx + "bn2_v"]),
                "conv3": P[pfx + "conv3"],
                "bn3": (P[pfx + "bn3_g"], P[pfx + "bn3_b"],
                        P[pfx + "bn3_m"], P[pfx + "bn3_v"]),
            }
            if (pfx + "down_conv") in P:
                blk["down_conv"] = P[pfx + "down_conv"]
                blk["down_bn"] = (P[pfx + "down_bn_g"], P[pfx + "down_bn_b"],
                                  P[pfx + "down_bn_m"], P[pfx + "down_bn_v"])
            x = _bottleneck(x, blk, s, _NB[(S, B == 0)])
    return _head(x, P["bneck_g"], P["bneck_b"], P["classifier"])


def kernel(bb_conv1, bb_bn1_g, bb_bn1_b, bb_bn1_m, bb_bn1_v, s0b0_conv1, s0b0_bn1_g, s0b0_bn1_b, s0b0_bn1_m, s0b0_bn1_v, s0b0_conv2, s0b0_bn2_g, s0b0_bn2_b, s0b0_bn2_m, s0b0_bn2_v, s0b0_conv3, s0b0_bn3_g, s0b0_bn3_b, s0b0_bn3_m, s0b0_bn3_v, s0b0_down_conv, s0b0_down_bn_g, s0b0_down_bn_b, s0b0_down_bn_m, s0b0_down_bn_v, s0b1_conv1, s0b1_bn1_g, s0b1_bn1_b, s0b1_bn1_m, s0b1_bn1_v, s0b1_conv2, s0b1_bn2_g, s0b1_bn2_b, s0b1_bn2_m, s0b1_bn2_v, s0b1_conv3, s0b1_bn3_g, s0b1_bn3_b, s0b1_bn3_m, s0b1_bn3_v, s0b2_conv1, s0b2_bn1_g, s0b2_bn1_b, s0b2_bn1_m, s0b2_bn1_v, s0b2_conv2, s0b2_bn2_g, s0b2_bn2_b, s0b2_bn2_m, s0b2_bn2_v, s0b2_conv3, s0b2_bn3_g, s0b2_bn3_b, s0b2_bn3_m, s0b2_bn3_v, s1b0_conv1, s1b0_bn1_g, s1b0_bn1_b, s1b0_bn1_m, s1b0_bn1_v, s1b0_conv2, s1b0_bn2_g, s1b0_bn2_b, s1b0_bn2_m, s1b0_bn2_v, s1b0_conv3, s1b0_bn3_g, s1b0_bn3_b, s1b0_bn3_m, s1b0_bn3_v, s1b0_down_conv, s1b0_down_bn_g, s1b0_down_bn_b, s1b0_down_bn_m, s1b0_down_bn_v, s1b1_conv1, s1b1_bn1_g, s1b1_bn1_b, s1b1_bn1_m, s1b1_bn1_v, s1b1_conv2, s1b1_bn2_g, s1b1_bn2_b, s1b1_bn2_m, s1b1_bn2_v, s1b1_conv3, s1b1_bn3_g, s1b1_bn3_b, s1b1_bn3_m, s1b1_bn3_v, s1b2_conv1, s1b2_bn1_g, s1b2_bn1_b, s1b2_bn1_m, s1b2_bn1_v, s1b2_conv2, s1b2_bn2_g, s1b2_bn2_b, s1b2_bn2_m, s1b2_bn2_v, s1b2_conv3, s1b2_bn3_g, s1b2_bn3_b, s1b2_bn3_m, s1b2_bn3_v, s1b3_conv1, s1b3_bn1_g, s1b3_bn1_b, s1b3_bn1_m, s1b3_bn1_v, s1b3_conv2, s1b3_bn2_g, s1b3_bn2_b, s1b3_bn2_m, s1b3_bn2_v, s1b3_conv3, s1b3_bn3_g, s1b3_bn3_b, s1b3_bn3_m, s1b3_bn3_v, s2b0_conv1, s2b0_bn1_g, s2b0_bn1_b, s2b0_bn1_m, s2b0_bn1_v, s2b0_conv2, s2b0_bn2_g, s2b0_bn2_b, s2b0_bn2_m, s2b0_bn2_v, s2b0_conv3, s2b0_bn3_g, s2b0_bn3_b, s2b0_bn3_m, s2b0_bn3_v, s2b0_down_conv, s2b0_down_bn_g, s2b0_down_bn_b, s2b0_down_bn_m, s2b0_down_bn_v, s2b1_conv1, s2b1_bn1_g, s2b1_bn1_b, s2b1_bn1_m, s2b1_bn1_v, s2b1_conv2, s2b1_bn2_g, s2b1_bn2_b, s2b1_bn2_m, s2b1_bn2_v, s2b1_conv3, s2b1_bn3_g, s2b1_bn3_b, s2b1_bn3_m, s2b1_bn3_v, s2b2_conv1, s2b2_bn1_g, s2b2_bn1_b, s2b2_bn1_m, s2b2_bn1_v, s2b2_conv2, s2b2_bn2_g, s2b2_bn2_b, s2b2_bn2_m, s2b2_bn2_v, s2b2_conv3, s2b2_bn3_g, s2b2_bn3_b, s2b2_bn3_m, s2b2_bn3_v, s2b3_conv1, s2b3_bn1_g, s2b3_bn1_b, s2b3_bn1_m, s2b3_bn1_v, s2b3_conv2, s2b3_bn2_g, s2b3_bn2_b, s2b3_bn2_m, s2b3_bn2_v, s2b3_conv3, s2b3_bn3_g, s2b3_bn3_b, s2b3_bn3_m, s2b3_bn3_v, s2b4_conv1, s2b4_bn1_g, s2b4_bn1_b, s2b4_bn1_m, s2b4_bn1_v, s2b4_conv2, s2b4_bn2_g, s2b4_bn2_b, s2b4_bn2_m, s2b4_bn2_v, s2b4_conv3, s2b4_bn3_g, s2b4_bn3_b, s2b4_bn3_m, s2b4_bn3_v, s2b5_conv1, s2b5_bn1_g, s2b5_bn1_b, s2b5_bn1_m, s2b5_bn1_v, s2b5_conv2, s2b5_bn2_g, s2b5_bn2_b, s2b5_bn2_m, s2b5_bn2_v, s2b5_conv3, s2b5_bn3_g, s2b5_bn3_b, s2b5_bn3_m, s2b5_bn3_v, s3b0_conv1, s3b0_bn1_g, s3b0_bn1_b, s3b0_bn1_m, s3b0_bn1_v, s3b0_conv2, s3b0_bn2_g, s3b0_bn2_b, s3b0_bn2_m, s3b0_bn2_v, s3b0_conv3, s3b0_bn3_g, s3b0_bn3_b, s3b0_bn3_m, s3b0_bn3_v, s3b0_down_conv, s3b0_down_bn_g, s3b0_down_bn_b, s3b0_down_bn_m, s3b0_down_bn_v, s3b1_conv1, s3b1_bn1_g, s3b1_bn1_b, s3b1_bn1_m, s3b1_bn1_v, s3b1_conv2, s3b1_bn2_g, s3b1_bn2_b, s3b1_bn2_m, s3b1_bn2_v, s3b1_conv3, s3b1_bn3_g, s3b1_bn3_b, s3b1_bn3_m, s3b1_bn3_v, s3b2_conv1, s3b2_bn1_g, s3b2_bn1_b, s3b2_bn1_m, s3b2_bn1_v, s3b2_conv2, s3b2_bn2_g, s3b2_bn2_b, s3b2_bn2_m, s3b2_bn2_v, s3b2_conv3, s3b2_bn3_g, s3b2_bn3_b, s3b2_bn3_m, s3b2_bn3_v, bneck_g, bneck_b, bneck_m, bneck_v, classifier, x):
    P = dict(locals())
    return _forward(P)


# speedup vs baseline: 1.0393x; 1.0393x over previous
import functools

import jax
import jax.numpy as jnp
from jax import lax
from jax.experimental import pallas as pl
from jax.experimental.pallas import tpu as pltpu

_EPS = 1e-5


def _bn_scale_shift(g, b, m, v):
    s = g * lax.rsqrt(v + _EPS)
    return s.astype(jnp.float32), (b - m * s).astype(jnp.float32)


def _dot_chunked(a, w, tk=512):
    K = a.shape[1]
    if K <= tk:
        return jnp.dot(a, w, preferred_element_type=jnp.float32)
    acc = jnp.dot(a[:, :tk], w[:tk], preferred_element_type=jnp.float32)
    for k0 in range(tk, K, tk):
        acc = acc + jnp.dot(a[:, k0:k0 + tk], w[k0:k0 + tk],
                            preferred_element_type=jnp.float32)
    return acc


def _dot_fori(lhs_slice, w_ref, M, N, K, tk):
    def body(k, acc):
        k0 = k * tk
        return acc + jnp.dot(lhs_slice(k0), w_ref[pl.ds(k0, tk)],
                             preferred_element_type=jnp.float32)
    return lax.fori_loop(0, K // tk, body, jnp.zeros((M, N), jnp.float32))


_CONV2_TK = {64: 128, 128: 384, 256: 384, 512: 512}


def _tap_s2(arr, i, j, Ho, Wo):
    nb, Hp, Wp, C = arr.shape
    r = arr.reshape(nb, Hp // 2, 2, Wp // 2, 2, C)
    return r[:, i // 2:i // 2 + Ho, i % 2, j // 2:j // 2 + Wo, j % 2, :]


def _stem_kernel(x_ref, w_ref, s_ref, t_ref, o_ref, scr_ref):
    v = x_ref[0].reshape(115, 2, 115, 6)
    for i in range(7):
        hs = v[i // 2:i // 2 + 112, i % 2]
        for k in range(4):
            w6 = min(6, 21 - 6 * k)
            piece = hs[:, k:k + 112, :w6].reshape(12544, w6)
            scr_ref[:, 21 * i + 6 * k:21 * i + 6 * k + w6] = piece
    scr_ref[:, 147:] = jnp.zeros((12544, 256 - 147), jnp.bfloat16)
    y = jnp.dot(scr_ref[...], w_ref[...], preferred_element_type=jnp.float32)
    y = jnp.maximum(y * s_ref[...] + t_ref[...], 0.0).astype(jnp.bfloat16)
    y = y.reshape(112, 112, 64)
    yp = jnp.pad(y, ((1, 1), (1, 1), (0, 0)),
                 constant_values=jnp.bfloat16(-jnp.inf))
    m = None
    for i in range(3):
        for j in range(3):
            tap = _tap_s2(yp[None], i, j, 56, 56)
            m = tap if m is None else jnp.maximum(m, tap)
    o_ref[...] = m


def _stem(x, w, g, b, mn, vr):
    N = x.shape[0]
    xf = x.reshape(N, 224, 672)
    xf = jnp.pad(xf, ((0, 0), (3, 3), (9, 9)))
    s, t = _bn_scale_shift(g, b, mn, vr)
    return pl.pallas_call(
        _stem_kernel,
        out_shape=jax.ShapeDtypeStruct((N, 56, 56, 64), jnp.bfloat16),
        grid=(N,),
        in_specs=[pl.BlockSpec((1, 230, 690), lambda n: (n, 0, 0)),
                  pl.BlockSpec((256, 64), lambda n: (0, 0)),
                  pl.BlockSpec((1, 64), lambda n: (0, 0)),
                  pl.BlockSpec((1, 64), lambda n: (0, 0))],
        out_specs=pl.BlockSpec((1, 56, 56, 64), lambda n: (n, 0, 0, 0)),
        scratch_shapes=[pltpu.VMEM((12544, 256), jnp.bfloat16)],
        compiler_params=pltpu.CompilerParams(
            dimension_semantics=("parallel",),
            vmem_limit_bytes=100 * 1024 * 1024),
    )(xf, w, s, t)


def _make_block_kernel(nb, H, W, cin, planes, stride, down):
    Ho, Wo = H // stride, W // stride
    M1 = nb * H * W
    Mo = nb * Ho * Wo
    cout = 4 * planes

    tk2 = _CONV2_TK.get(planes, 9 * planes)
    Kp2 = -(-9 * planes // tk2) * tk2

    def kern(x_ref, w1_ref, s1_ref, t1_ref, w2_ref, s2_ref, t2_ref,
             w3_ref, s3_ref, t3_ref, *rest):
        if down:
            wd_ref, sd_ref, td_ref, o_ref, scr_ref = rest
        else:
            o_ref, scr_ref = rest
        x4 = x_ref[...]
        x = x4.reshape(M1, cin)

        if cin > 512:
            u = _dot_fori(
                lambda k0: x_ref[:, :, :, pl.ds(k0, 512)].reshape(M1, 512),
                w1_ref, M1, planes, cin, 512)
        else:
            u = jnp.dot(x, w1_ref[...], preferred_element_type=jnp.float32)
        u = jnp.maximum(u * s1_ref[...] + t1_ref[...], 0.0).astype(jnp.bfloat16)

        up = jnp.pad(u.reshape(nb, H, W, planes),
                     ((0, 0), (1, 1), (1, 1), (0, 0)))
        for i in range(3):
            for j in range(3):
                if stride == 1:
                    a = up[:, i:i + Ho, j:j + Wo, :]
                else:
                    a = _tap_s2(up, i, j, Ho, Wo)
                t = i * 3 + j
                scr_ref[:, t * planes:(t + 1) * planes] = a.reshape(Mo, planes)
        if Kp2 > 9 * planes:
            scr_ref[:, 9 * planes:] = jnp.zeros((Mo, Kp2 - 9 * planes),
                                                jnp.bfloat16)
        acc = _dot_fori(lambda k0: scr_ref[:, pl.ds(k0, tk2)],
                        w2_ref, Mo, planes, Kp2, tk2)
        v = jnp.maximum(acc * s2_ref[...] + t2_ref[...], 0.0).astype(jnp.bfloat16)

        y = jnp.dot(v, w3_ref[...], preferred_element_type=jnp.float32)
        y = (y * s3_ref[...] + t3_ref[...]).astype(jnp.bfloat16)

        if down:
            if cin > 512:
                def lhsd(k0):
                    v = x_ref[:, :, :, pl.ds(k0, 512)]
                    if stride != 1:
                        v = _tap_s2(v, 0, 0, Ho, Wo)
                    return v.reshape(Mo, 512)
                idn = _dot_fori(lhsd, wd_ref, Mo, cout, cin, 512)
            else:
                if stride == 1:
                    xs = x
                else:
                    xs = _tap_s2(x4, 0, 0, Ho, Wo).reshape(Mo, cin)
                idn = jnp.dot(xs, wd_ref[...],
                              preferred_element_type=jnp.float32)
            idn = (idn * sd_ref[...] + td_ref[...]).astype(jnp.bfloat16)
        else:
            idn = x.reshape(Mo, cout)

        out = jnp.maximum(y.astype(jnp.float32) + idn.astype(jnp.float32), 0.0)
        o_ref[...] = out.astype(jnp.bfloat16).reshape(nb, Ho, Wo, cout)

    return kern


def _bottleneck(x, p, stride, nb):
    N, H, W, cin = x.shape
    w1 = p["conv1"]
    planes = w1.shape[1]
    cout = 4 * planes
    Ho, Wo = H // stride, W // stride
    down = "down_conv" in p

    s1, t1 = _bn_scale_shift(*p["bn1"])
    s2, t2 = _bn_scale_shift(*p["bn2"])
    s3, t3 = _bn_scale_shift(*p["bn3"])
    w2 = p["conv2"]
    Kp2 = w2.shape[0]
    w3 = p["conv3"]

    c = pl.BlockSpec
    in_specs = [
        c((nb, H, W, cin), lambda n: (n, 0, 0, 0)),
        c((cin, planes), lambda n: (0, 0)),
        c((1, planes), lambda n: (0, 0)),
        c((1, planes), lambda n: (0, 0)),
        c((Kp2, planes), lambda n: (0, 0)),
        c((1, planes), lambda n: (0, 0)),
        c((1, planes), lambda n: (0, 0)),
        c((planes, cout), lambda n: (0, 0)),
        c((1, cout), lambda n: (0, 0)),
        c((1, cout), lambda n: (0, 0)),
    ]
    args = [x, w1, s1, t1, w2, s2, t2, w3, s3, t3]
    if down:
        sd, td = _bn_scale_shift(*p["down_bn"])
        in_specs += [c((cin, cout), lambda n: (0, 0)),
                     c((1, cout), lambda n: (0, 0)),
                     c((1, cout), lambda n: (0, 0))]
        args += [p["down_conv"], sd, td]

    kern = _make_block_kernel(nb, H, W, cin, planes, stride, down)
    return pl.pallas_call(
        kern,
        out_shape=jax.ShapeDtypeStruct((N, Ho, Wo, cout), jnp.bfloat16),
        grid=(N // nb,),
        in_specs=in_specs,
        out_specs=c((nb, Ho, Wo, cout), lambda n: (n, 0, 0, 0)),
        scratch_shapes=[pltpu.VMEM((nb * Ho * Wo, Kp2), jnp.bfloat16)],
        compiler_params=pltpu.CompilerParams(
            dimension_semantics=("parallel",),
            vmem_limit_bytes=100 * 1024 * 1024),
    )(*args)


def _head_kernel(f_ref, g_ref, b_ref, w_ref, cls_ref, feat_ref, scr_ref):
    f = f_ref[...].astype(jnp.float32)
    gfeat = jnp.mean(f, axis=1)
    mean = jnp.mean(gfeat, axis=0, keepdims=True)
    var = jnp.mean(jnp.square(gfeat - mean), axis=0, keepdims=True)
    bnf = (gfeat - mean) * lax.rsqrt(var + _EPS) * g_ref[...] + b_ref[...]
    scr_ref[...] = bnf.astype(jnp.bfloat16)
    N, C = scr_ref.shape
    cls_ref[...] = _dot_fori(lambda k0: scr_ref[:, pl.ds(k0, 512)],
                             w_ref, N, w_ref.shape[1], C, 512)
    feat_ref[...] = gfeat


def _head(feat, g, b, w):
    N = feat.shape[0]
    C = feat.shape[3]
    nc = w.shape[1]
    f3 = feat.reshape(N, 49, C)
    return pl.pallas_call(
        _head_kernel,
        out_shape=(jax.ShapeDtypeStruct((N, nc), jnp.float32),
                   jax.ShapeDtypeStruct((N, C), jnp.float32)),
        grid=(1,),
        in_specs=[pl.BlockSpec((N, 49, C), lambda i: (0, 0, 0)),
                  pl.BlockSpec((1, C), lambda i: (0, 0)),
                  pl.BlockSpec((1, C), lambda i: (0, 0)),
                  pl.BlockSpec((C, nc), lambda i: (0, 0))],
        out_specs=[pl.BlockSpec((N, nc), lambda i: (0, 0)),
                   pl.BlockSpec((N, C), lambda i: (0, 0))],
        scratch_shapes=[pltpu.VMEM((N, C), jnp.bfloat16)],
        compiler_params=pltpu.CompilerParams(
            vmem_limit_bytes=100 * 1024 * 1024),
    )(f3, g, b, w)


_STAGES = [(64, 3, 1), (128, 4, 2), (256, 6, 2), (512, 3, 2)]
_NB = {(0, True): 2, (0, False): 2,
       (1, True): 2, (1, False): 4,
       (2, True): 4, (2, False): 8,
       (3, True): 8, (3, False): 8}


@jax.jit
def _forward(P):
    x = jnp.transpose(P["x"], (0, 2, 3, 1)).astype(jnp.bfloat16)
    x = _stem(x, P["bb_conv1"], P["bb_bn1_g"], P["bb_bn1_b"],
              P["bb_bn1_m"], P["bb_bn1_v"])
    for S, (planes, nblocks, stride) in enumerate(_STAGES):
        for B in range(nblocks):
            s = stride if B == 0 else 1
            pfx = "s%db%d_" % (S, B)
            blk = {
                "conv1": P[pfx + "conv1"],
                "bn1": (P[pfx + "bn1_g"], P[pfx + "bn1_b"],
                        P[pfx + "bn1_m"], P[pfx + "bn1_v"]),
                "conv2": P[pfx + "conv2"],
                "bn2": (P[pfx + "bn2_g"], P[pfx + "bn2_b"],
                        P[pfx + "bn2_m"], P[pfx + "bn2_v"]),
                "conv3": P[pfx + "conv3"],
                "bn3": (P[pfx + "bn3_g"], P[pfx + "bn3_b"],
                        P[pfx + "bn3_m"], P[pfx + "bn3_v"]),
            }
            if (pfx + "down_conv") in P:
                blk["down_conv"] = P[pfx + "down_conv"]
                blk["down_bn"] = (P[pfx + "down_bn_g"], P[pfx + "down_bn_b"],
                                  P[pfx + "down_bn_m"], P[pfx + "down_bn_v"])
            x = _bottleneck(x, blk, s, _NB[(S, B == 0)])
    return _head(x, P["bneck_g"], P["bneck_b"], P["classifier"])


def kernel(bb_conv1, bb_bn1_g, bb_bn1_b, bb_bn1_m, bb_bn1_v, s0b0_conv1, s0b0_bn1_g, s0b0_bn1_b, s0b0_bn1_m, s0b0_bn1_v, s0b0_conv2, s0b0_bn2_g, s0b0_bn2_b, s0b0_bn2_m, s0b0_bn2_v, s0b0_conv3, s0b0_bn3_g, s0b0_bn3_b, s0b0_bn3_m, s0b0_bn3_v, s0b0_down_conv, s0b0_down_bn_g, s0b0_down_bn_b, s0b0_down_bn_m, s0b0_down_bn_v, s0b1_conv1, s0b1_bn1_g, s0b1_bn1_b, s0b1_bn1_m, s0b1_bn1_v, s0b1_conv2, s0b1_bn2_g, s0b1_bn2_b, s0b1_bn2_m, s0b1_bn2_v, s0b1_conv3, s0b1_bn3_g, s0b1_bn3_b, s0b1_bn3_m, s0b1_bn3_v, s0b2_conv1, s0b2_bn1_g, s0b2_bn1_b, s0b2_bn1_m, s0b2_bn1_v, s0b2_conv2, s0b2_bn2_g, s0b2_bn2_b, s0b2_bn2_m, s0b2_bn2_v, s0b2_conv3, s0b2_bn3_g, s0b2_bn3_b, s0b2_bn3_m, s0b2_bn3_v, s1b0_conv1, s1b0_bn1_g, s1b0_bn1_b, s1b0_bn1_m, s1b0_bn1_v, s1b0_conv2, s1b0_bn2_g, s1b0_bn2_b, s1b0_bn2_m, s1b0_bn2_v, s1b0_conv3, s1b0_bn3_g, s1b0_bn3_b, s1b0_bn3_m, s1b0_bn3_v, s1b0_down_conv, s1b0_down_bn_g, s1b0_down_bn_b, s1b0_down_bn_m, s1b0_down_bn_v, s1b1_conv1, s1b1_bn1_g, s1b1_bn1_b, s1b1_bn1_m, s1b1_bn1_v, s1b1_conv2, s1b1_bn2_g, s1b1_bn2_b, s1b1_bn2_m, s1b1_bn2_v, s1b1_conv3, s1b1_bn3_g, s1b1_bn3_b, s1b1_bn3_m, s1b1_bn3_v, s1b2_conv1, s1b2_bn1_g, s1b2_bn1_b, s1b2_bn1_m, s1b2_bn1_v, s1b2_conv2, s1b2_bn2_g, s1b2_bn2_b, s1b2_bn2_m, s1b2_bn2_v, s1b2_conv3, s1b2_bn3_g, s1b2_bn3_b, s1b2_bn3_m, s1b2_bn3_v, s1b3_conv1, s1b3_bn1_g, s1b3_bn1_b, s1b3_bn1_m, s1b3_bn1_v, s1b3_conv2, s1b3_bn2_g, s1b3_bn2_b, s1b3_bn2_m, s1b3_bn2_v, s1b3_conv3, s1b3_bn3_g, s1b3_bn3_b, s1b3_bn3_m, s1b3_bn3_v, s2b0_conv1, s2b0_bn1_g, s2b0_bn1_b, s2b0_bn1_m, s2b0_bn1_v, s2b0_conv2, s2b0_bn2_g, s2b0_bn2_b, s2b0_bn2_m, s2b0_bn2_v, s2b0_conv3, s2b0_bn3_g, s2b0_bn3_b, s2b0_bn3_m, s2b0_bn3_v, s2b0_down_conv, s2b0_down_bn_g, s2b0_down_bn_b, s2b0_down_bn_m, s2b0_down_bn_v, s2b1_conv1, s2b1_bn1_g, s2b1_bn1_b, s2b1_bn1_m, s2b1_bn1_v, s2b1_conv2, s2b1_bn2_g, s2b1_bn2_b, s2b1_bn2_m, s2b1_bn2_v, s2b1_conv3, s2b1_bn3_g, s2b1_bn3_b, s2b1_bn3_m, s2b1_bn3_v, s2b2_conv1, s2b2_bn1_g, s2b2_bn1_b, s2b2_bn1_m, s2b2_bn1_v, s2b2_conv2, s2b2_bn2_g, s2b2_bn2_b, s2b2_bn2_m, s2b2_bn2_v, s2b2_conv3, s2b2_bn3_g, s2b2_bn3_b, s2b2_bn3_m, s2b2_bn3_v, s2b3_conv1, s2b3_bn1_g, s2b3_bn1_b, s2b3_bn1_m, s2b3_bn1_v, s2b3_conv2, s2b3_bn2_g, s2b3_bn2_b, s2b3_bn2_m, s2b3_bn2_v, s2b3_conv3, s2b3_bn3_g, s2b3_bn3_b, s2b3_bn3_m, s2b3_bn3_v, s2b4_conv1, s2b4_bn1_g, s2b4_bn1_b, s2b4_bn1_m, s2b4_bn1_v, s2b4_conv2, s2b4_bn2_g, s2b4_bn2_b, s2b4_bn2_m, s2b4_bn2_v, s2b4_conv3, s2b4_bn3_g, s2b4_bn3_b, s2b4_bn3_m, s2b4_bn3_v, s2b5_conv1, s2b5_bn1_g, s2b5_bn1_b, s2b5_bn1_m, s2b5_bn1_v, s2b5_conv2, s2b5_bn2_g, s2b5_bn2_b, s2b5_bn2_m, s2b5_bn2_v, s2b5_conv3, s2b5_bn3_g, s2b5_bn3_b, s2b5_bn3_m, s2b5_bn3_v, s3b0_conv1, s3b0_bn1_g, s3b0_bn1_b, s3b0_bn1_m, s3b0_bn1_v, s3b0_conv2, s3b0_bn2_g, s3b0_bn2_b, s3b0_bn2_m, s3b0_bn2_v, s3b0_conv3, s3b0_bn3_g, s3b0_bn3_b, s3b0_bn3_m, s3b0_bn3_v, s3b0_down_conv, s3b0_down_bn_g, s3b0_down_bn_b, s3b0_down_bn_m, s3b0_down_bn_v, s3b1_conv1, s3b1_bn1_g, s3b1_bn1_b, s3b1_bn1_m, s3b1_bn1_v, s3b1_conv2, s3b1_bn2_g, s3b1_bn2_b, s3b1_bn2_m, s3b1_bn2_v, s3b1_conv3, s3b1_bn3_g, s3b1_bn3_b, s3b1_bn3_m, s3b1_bn3_v, s3b2_conv1, s3b2_bn1_g, s3b2_bn1_b, s3b2_bn1_m, s3b2_bn1_v, s3b2_conv2, s3b2_bn2_g, s3b2_bn2_b, s3b2_bn2_m, s3b2_bn2_v, s3b2_conv3, s3b2_bn3_g, s3b2_bn3_b, s3b2_bn3_m, s3b2_bn3_v, bneck_g, bneck_b, bneck_m, bneck_v, classifier, x):
    P = dict(locals())
    return _forward(P)


# speedup vs baseline: 1.5187x; 1.4613x over previous
import functools

import jax
import jax.numpy as jnp
from jax import lax
from jax.experimental import pallas as pl
from jax.experimental.pallas import tpu as pltpu

_EPS = 1e-5


def _bn_scale_shift(g, b, m, v):
    s = g * lax.rsqrt(v + _EPS)
    return s.astype(jnp.float32), (b - m * s).astype(jnp.float32)


def _dot_chunked(a, w, tk=512):
    K = a.shape[1]
    if K <= tk:
        return jnp.dot(a, w, preferred_element_type=jnp.float32)
    acc = jnp.dot(a[:, :tk], w[:tk], preferred_element_type=jnp.float32)
    for k0 in range(tk, K, tk):
        acc = acc + jnp.dot(a[:, k0:k0 + tk], w[k0:k0 + tk],
                            preferred_element_type=jnp.float32)
    return acc


def _dot_fori(lhs_slice, w_ref, M, N, K, tk):
    def body(k, acc):
        k0 = k * tk
        return acc + jnp.dot(lhs_slice(k0), w_ref[pl.ds(k0, tk)],
                             preferred_element_type=jnp.float32)
    return lax.fori_loop(0, K // tk, body, jnp.zeros((M, N), jnp.float32))


_CONV2_TK = {64: 128, 128: 384, 256: 384, 512: 512}


def _tap_s2(arr, i, j, Ho, Wo):
    nb, Hp, Wp, C = arr.shape
    r = arr.reshape(nb, Hp // 2, 2, Wp // 2, 2, C)
    return r[:, i // 2:i // 2 + Ho, i % 2, j // 2:j // 2 + Wo, j % 2, :]


def _stem_kernel(x_ref, w_ref, s_ref, t_ref, o_ref, scr_ref):
    v = x_ref[0].reshape(115, 2, 115, 6)
    for i in range(7):
        hs = v[i // 2:i // 2 + 112, i % 2]
        stripe = jnp.concatenate(
            [hs[:, k:k + 112, :min(6, 21 - 6 * k)] for k in range(4)],
            axis=-1)
        scr_ref[:, 21 * i:21 * i + 21] = stripe.reshape(12544, 21)
    scr_ref[:, 147:] = jnp.zeros((12544, 256 - 147), jnp.bfloat16)
    y = jnp.dot(scr_ref[...], w_ref[...], preferred_element_type=jnp.float32)
    y = jnp.maximum(y * s_ref[...] + t_ref[...], 0.0).astype(jnp.bfloat16)
    y = y.reshape(112, 112, 64)
    yp = jnp.pad(y, ((1, 1), (1, 1), (0, 0)),
                 constant_values=jnp.bfloat16(-jnp.inf))
    m = None
    for i in range(3):
        for j in range(3):
            tap = _tap_s2(yp[None], i, j, 56, 56)
            m = tap if m is None else jnp.maximum(m, tap)
    o_ref[...] = m


def _stem(x, w, g, b, mn, vr):
    N = x.shape[0]
    xf = x.reshape(N, 224, 672)
    xf = jnp.pad(xf, ((0, 0), (3, 3), (9, 9)))
    s, t = _bn_scale_shift(g, b, mn, vr)
    return pl.pallas_call(
        _stem_kernel,
        out_shape=jax.ShapeDtypeStruct((N, 56, 56, 64), jnp.bfloat16),
        grid=(N,),
        in_specs=[pl.BlockSpec((1, 230, 690), lambda n: (n, 0, 0)),
                  pl.BlockSpec((256, 64), lambda n: (0, 0)),
                  pl.BlockSpec((1, 64), lambda n: (0, 0)),
                  pl.BlockSpec((1, 64), lambda n: (0, 0))],
        out_specs=pl.BlockSpec((1, 56, 56, 64), lambda n: (n, 0, 0, 0)),
        scratch_shapes=[pltpu.VMEM((12544, 256), jnp.bfloat16)],
        compiler_params=pltpu.CompilerParams(
            dimension_semantics=("parallel",),
            vmem_limit_bytes=100 * 1024 * 1024),
    )(xf, w, s, t)


def _make_block_kernel(nb, H, W, cin, planes, stride, down):
    Ho, Wo = H // stride, W // stride
    M1 = nb * H * W
    Mo = nb * Ho * Wo
    cout = 4 * planes

    tk2 = _CONV2_TK.get(planes, 9 * planes)
    Kp2 = -(-9 * planes // tk2) * tk2

    def kern(x_ref, w1_ref, s1_ref, t1_ref, w2_ref, s2_ref, t2_ref,
             w3_ref, s3_ref, t3_ref, *rest):
        if down:
            wd_ref, sd_ref, td_ref, o_ref, scr_ref = rest
        else:
            o_ref, scr_ref = rest
        x4 = x_ref[...]
        x = x4.reshape(M1, cin)

        if cin > 512:
            u = _dot_fori(
                lambda k0: x_ref[:, :, :, pl.ds(k0, 512)].reshape(M1, 512),
                w1_ref, M1, planes, cin, 512)
        else:
            u = jnp.dot(x, w1_ref[...], preferred_element_type=jnp.float32)
        u = jnp.maximum(u * s1_ref[...] + t1_ref[...], 0.0).astype(jnp.bfloat16)

        up = jnp.pad(u.reshape(nb, H, W, planes),
                     ((0, 0), (1, 1), (1, 1), (0, 0)))
        for i in range(3):
            for j in range(3):
                if stride == 1:
                    a = up[:, i:i + Ho, j:j + Wo, :]
                else:
                    a = _tap_s2(up, i, j, Ho, Wo)
                t = i * 3 + j
                scr_ref[:, t * planes:(t + 1) * planes] = a.reshape(Mo, planes)
        if Kp2 > 9 * planes:
            scr_ref[:, 9 * planes:] = jnp.zeros((Mo, Kp2 - 9 * planes),
                                                jnp.bfloat16)
        acc = _dot_fori(lambda k0: scr_ref[:, pl.ds(k0, tk2)],
                        w2_ref, Mo, planes, Kp2, tk2)
        v = jnp.maximum(acc * s2_ref[...] + t2_ref[...], 0.0).astype(jnp.bfloat16)

        y = jnp.dot(v, w3_ref[...], preferred_element_type=jnp.float32)
        y = (y * s3_ref[...] + t3_ref[...]).astype(jnp.bfloat16)

        if down:
            if cin > 512:
                def lhsd(k0):
                    v = x_ref[:, :, :, pl.ds(k0, 512)]
                    if stride != 1:
                        v = _tap_s2(v, 0, 0, Ho, Wo)
                    return v.reshape(Mo, 512)
                idn = _dot_fori(lhsd, wd_ref, Mo, cout, cin, 512)
            else:
                if stride == 1:
                    xs = x
                else:
                    xs = _tap_s2(x4, 0, 0, Ho, Wo).reshape(Mo, cin)
                idn = jnp.dot(xs, wd_ref[...],
                              preferred_element_type=jnp.float32)
            idn = (idn * sd_ref[...] + td_ref[...]).astype(jnp.bfloat16)
        else:
            idn = x.reshape(Mo, cout)

        out = jnp.maximum(y.astype(jnp.float32) + idn.astype(jnp.float32), 0.0)
        o_ref[...] = out.astype(jnp.bfloat16).reshape(nb, Ho, Wo, cout)

    return kern


def _bottleneck(x, p, stride, nb):
    N, H, W, cin = x.shape
    w1 = p["conv1"]
    planes = w1.shape[1]
    cout = 4 * planes
    Ho, Wo = H // stride, W // stride
    down = "down_conv" in p

    s1, t1 = _bn_scale_shift(*p["bn1"])
    s2, t2 = _bn_scale_shift(*p["bn2"])
    s3, t3 = _bn_scale_shift(*p["bn3"])
    w2 = p["conv2"]
    Kp2 = w2.shape[0]
    w3 = p["conv3"]

    c = pl.BlockSpec
    in_specs = [
        c((nb, H, W, cin), lambda n: (n, 0, 0, 0)),
        c((cin, planes), lambda n: (0, 0)),
        c((1, planes), lambda n: (0, 0)),
        c((1, planes), lambda n: (0, 0)),
        c((Kp2, planes), lambda n: (0, 0)),
        c((1, planes), lambda n: (0, 0)),
        c((1, planes), lambda n: (0, 0)),
        c((planes, cout), lambda n: (0, 0)),
        c((1, cout), lambda n: (0, 0)),
        c((1, cout), lambda n: (0, 0)),
    ]
    args = [x, w1, s1, t1, w2, s2, t2, w3, s3, t3]
    if down:
        sd, td = _bn_scale_shift(*p["down_bn"])
        in_specs += [c((cin, cout), lambda n: (0, 0)),
                     c((1, cout), lambda n: (0, 0)),
                     c((1, cout), lambda n: (0, 0))]
        args += [p["down_conv"], sd, td]

    kern = _make_block_kernel(nb, H, W, cin, planes, stride, down)
    return pl.pallas_call(
        kern,
        out_shape=jax.ShapeDtypeStruct((N, Ho, Wo, cout), jnp.bfloat16),
        grid=(N // nb,),
        in_specs=in_specs,
        out_specs=c((nb, Ho, Wo, cout), lambda n: (n, 0, 0, 0)),
        scratch_shapes=[pltpu.VMEM((nb * Ho * Wo, Kp2), jnp.bfloat16)],
        compiler_params=pltpu.CompilerParams(
            dimension_semantics=("parallel",),
            vmem_limit_bytes=100 * 1024 * 1024),
    )(*args)


def _head_kernel(f_ref, g_ref, b_ref, w_ref, cls_ref, feat_ref, scr_ref):
    f = f_ref[...].astype(jnp.float32)
    gfeat = jnp.mean(f, axis=1)
    mean = jnp.mean(gfeat, axis=0, keepdims=True)
    var = jnp.mean(jnp.square(gfeat - mean), axis=0, keepdims=True)
    bnf = (gfeat - mean) * lax.rsqrt(var + _EPS) * g_ref[...] + b_ref[...]
    scr_ref[...] = bnf.astype(jnp.bfloat16)
    N, C = scr_ref.shape
    cls_ref[...] = _dot_fori(lambda k0: scr_ref[:, pl.ds(k0, 512)],
                             w_ref, N, w_ref.shape[1], C, 512)
    feat_ref[...] = gfeat


def _head(feat, g, b, w):
    N = feat.shape[0]
    C = feat.shape[3]
    nc = w.shape[1]
    f3 = feat.reshape(N, 49, C)
    return pl.pallas_call(
        _head_kernel,
        out_shape=(jax.ShapeDtypeStruct((N, nc), jnp.float32),
                   jax.ShapeDtypeStruct((N, C), jnp.float32)),
        grid=(1,),
        in_specs=[pl.BlockSpec((N, 49, C), lambda i: (0, 0, 0)),
                  pl.BlockSpec((1, C), lambda i: (0, 0)),
                  pl.BlockSpec((1, C), lambda i: (0, 0)),
                  pl.BlockSpec((C, nc), lambda i: (0, 0))],
        out_specs=[pl.BlockSpec((N, nc), lambda i: (0, 0)),
                   pl.BlockSpec((N, C), lambda i: (0, 0))],
        scratch_shapes=[pltpu.VMEM((N, C), jnp.bfloat16)],
        compiler_params=pltpu.CompilerParams(
            vmem_limit_bytes=100 * 1024 * 1024),
    )(f3, g, b, w)


_STAGES = [(64, 3, 1), (128, 4, 2), (256, 6, 2), (512, 3, 2)]
_NB = {(0, True): 2, (0, False): 2,
       (1, True): 2, (1, False): 4,
       (2, True): 4, (2, False): 8,
       (3, True): 8, (3, False): 8}


@jax.jit
def _forward(P):
    x = jnp.transpose(P["x"], (0, 2, 3, 1)).astype(jnp.bfloat16)
    x = _stem(x, P["bb_conv1"], P["bb_bn1_g"], P["bb_bn1_b"],
              P["bb_bn1_m"], P["bb_bn1_v"])
    for S, (planes, nblocks, stride) in enumerate(_STAGES):
        for B in range(nblocks):
            s = stride if B == 0 else 1
            pfx = "s%db%d_" % (S, B)
            blk = {
                "conv1": P[pfx + "conv1"],
                "bn1": (P[pfx + "bn1_g"], P[pfx + "bn1_b"],
                        P[pfx + "bn1_m"], P[pfx + "bn1_v"]),
                "conv2": P[pfx + "conv2"],
                "bn2": (P[pfx + "bn2_g"], P[pfx + "bn2_b"],
                        P[pfx + "bn2_m"], P[pfx + "bn2_v"]),
                "conv3": P[pfx + "conv3"],
                "bn3": (P[pfx + "bn3_g"], P[pfx + "bn3_b"],
                        P[pfx + "bn3_m"], P[pfx + "bn3_v"]),
            }
            if (pfx + "down_conv") in P:
                blk["down_conv"] = P[pfx + "down_conv"]
                blk["down_bn"] = (P[pfx + "down_bn_g"], P[pfx + "down_bn_b"],
                                  P[pfx + "down_bn_m"], P[pfx + "down_bn_v"])
            x = _bottleneck(x, blk, s, _NB[(S, B == 0)])
    return _head(x, P["bneck_g"], P["bneck_b"], P["classifier"])


def kernel(bb_conv1, bb_bn1_g, bb_bn1_b, bb_bn1_m, bb_bn1_v, s0b0_conv1, s0b0_bn1_g, s0b0_bn1_b, s0b0_bn1_m, s0b0_bn1_v, s0b0_conv2, s0b0_bn2_g, s0b0_bn2_b, s0b0_bn2_m, s0b0_bn2_v, s0b0_conv3, s0b0_bn3_g, s0b0_bn3_b, s0b0_bn3_m, s0b0_bn3_v, s0b0_down_conv, s0b0_down_bn_g, s0b0_down_bn_b, s0b0_down_bn_m, s0b0_down_bn_v, s0b1_conv1, s0b1_bn1_g, s0b1_bn1_b, s0b1_bn1_m, s0b1_bn1_v, s0b1_conv2, s0b1_bn2_g, s0b1_bn2_b, s0b1_bn2_m, s0b1_bn2_v, s0b1_conv3, s0b1_bn3_g, s0b1_bn3_b, s0b1_bn3_m, s0b1_bn3_v, s0b2_conv1, s0b2_bn1_g, s0b2_bn1_b, s0b2_bn1_m, s0b2_bn1_v, s0b2_conv2, s0b2_bn2_g, s0b2_bn2_b, s0b2_bn2_m, s0b2_bn2_v, s0b2_conv3, s0b2_bn3_g, s0b2_bn3_b, s0b2_bn3_m, s0b2_bn3_v, s1b0_conv1, s1b0_bn1_g, s1b0_bn1_b, s1b0_bn1_m, s1b0_bn1_v, s1b0_conv2, s1b0_bn2_g, s1b0_bn2_b, s1b0_bn2_m, s1b0_bn2_v, s1b0_conv3, s1b0_bn3_g, s1b0_bn3_b, s1b0_bn3_m, s1b0_bn3_v, s1b0_down_conv, s1b0_down_bn_g, s1b0_down_bn_b, s1b0_down_bn_m, s1b0_down_bn_v, s1b1_conv1, s1b1_bn1_g, s1b1_bn1_b, s1b1_bn1_m, s1b1_bn1_v, s1b1_conv2, s1b1_bn2_g, s1b1_bn2_b, s1b1_bn2_m, s1b1_bn2_v, s1b1_conv3, s1b1_bn3_g, s1b1_bn3_b, s1b1_bn3_m, s1b1_bn3_v, s1b2_conv1, s1b2_bn1_g, s1b2_bn1_b, s1b2_bn1_m, s1b2_bn1_v, s1b2_conv2, s1b2_bn2_g, s1b2_bn2_b, s1b2_bn2_m, s1b2_bn2_v, s1b2_conv3, s1b2_bn3_g, s1b2_bn3_b, s1b2_bn3_m, s1b2_bn3_v, s1b3_conv1, s1b3_bn1_g, s1b3_bn1_b, s1b3_bn1_m, s1b3_bn1_v, s1b3_conv2, s1b3_bn2_g, s1b3_bn2_b, s1b3_bn2_m, s1b3_bn2_v, s1b3_conv3, s1b3_bn3_g, s1b3_bn3_b, s1b3_bn3_m, s1b3_bn3_v, s2b0_conv1, s2b0_bn1_g, s2b0_bn1_b, s2b0_bn1_m, s2b0_bn1_v, s2b0_conv2, s2b0_bn2_g, s2b0_bn2_b, s2b0_bn2_m, s2b0_bn2_v, s2b0_conv3, s2b0_bn3_g, s2b0_bn3_b, s2b0_bn3_m, s2b0_bn3_v, s2b0_down_conv, s2b0_down_bn_g, s2b0_down_bn_b, s2b0_down_bn_m, s2b0_down_bn_v, s2b1_conv1, s2b1_bn1_g, s2b1_bn1_b, s2b1_bn1_m, s2b1_bn1_v, s2b1_conv2, s2b1_bn2_g, s2b1_bn2_b, s2b1_bn2_m, s2b1_bn2_v, s2b1_conv3, s2b1_bn3_g, s2b1_bn3_b, s2b1_bn3_m, s2b1_bn3_v, s2b2_conv1, s2b2_bn1_g, s2b2_bn1_b, s2b2_bn1_m, s2b2_bn1_v, s2b2_conv2, s2b2_bn2_g, s2b2_bn2_b, s2b2_bn2_m, s2b2_bn2_v, s2b2_conv3, s2b2_bn3_g, s2b2_bn3_b, s2b2_bn3_m, s2b2_bn3_v, s2b3_conv1, s2b3_bn1_g, s2b3_bn1_b, s2b3_bn1_m, s2b3_bn1_v, s2b3_conv2, s2b3_bn2_g, s2b3_bn2_b, s2b3_bn2_m, s2b3_bn2_v, s2b3_conv3, s2b3_bn3_g, s2b3_bn3_b, s2b3_bn3_m, s2b3_bn3_v, s2b4_conv1, s2b4_bn1_g, s2b4_bn1_b, s2b4_bn1_m, s2b4_bn1_v, s2b4_conv2, s2b4_bn2_g, s2b4_bn2_b, s2b4_bn2_m, s2b4_bn2_v, s2b4_conv3, s2b4_bn3_g, s2b4_bn3_b, s2b4_bn3_m, s2b4_bn3_v, s2b5_conv1, s2b5_bn1_g, s2b5_bn1_b, s2b5_bn1_m, s2b5_bn1_v, s2b5_conv2, s2b5_bn2_g, s2b5_bn2_b, s2b5_bn2_m, s2b5_bn2_v, s2b5_conv3, s2b5_bn3_g, s2b5_bn3_b, s2b5_bn3_m, s2b5_bn3_v, s3b0_conv1, s3b0_bn1_g, s3b0_bn1_b, s3b0_bn1_m, s3b0_bn1_v, s3b0_conv2, s3b0_bn2_g, s3b0_bn2_b, s3b0_bn2_m, s3b0_bn2_v, s3b0_conv3, s3b0_bn3_g, s3b0_bn3_b, s3b0_bn3_m, s3b0_bn3_v, s3b0_down_conv, s3b0_down_bn_g, s3b0_down_bn_b, s3b0_down_bn_m, s3b0_down_bn_v, s3b1_conv1, s3b1_bn1_g, s3b1_bn1_b, s3b1_bn1_m, s3b1_bn1_v, s3b1_conv2, s3b1_bn2_g, s3b1_bn2_b, s3b1_bn2_m, s3b1_bn2_v, s3b1_conv3, s3b1_bn3_g, s3b1_bn3_b, s3b1_bn3_m, s3b1_bn3_v, s3b2_conv1, s3b2_bn1_g, s3b2_bn1_b, s3b2_bn1_m, s3b2_bn1_v, s3b2_conv2, s3b2_bn2_g, s3b2_bn2_b, s3b2_bn2_m, s3b2_bn2_v, s3b2_conv3, s3b2_bn3_g, s3b2_bn3_b, s3b2_bn3_m, s3b2_bn3_v, bneck_g, bneck_b, bneck_m, bneck_v, classifier, x):
    P = dict(locals())
    return _forward(P)
